# Initial kernel scaffold; baseline (speedup 1.0000x reference)
import jax
import jax.numpy as jnp
from jax.experimental import pallas as pl


def kernel(inputs, row_indices, col_indices, edge_rel_sorted, conv_weights, attn_l, attn_r, h_bias):
    raise NotImplementedError("write your pallas kernel here")


# speedup vs baseline: 34.5214x; 34.5214x over previous
import jax
import jax.numpy as jnp
from jax import lax
from jax.experimental import pallas as pl
from jax.experimental.pallas import tpu as pltpu
from jax.experimental.pallas import tpu_sc as plsc

N = 10000
E = 160000
R = 8
H = 8
IN = 256
DH = 32
SLOPE = 0.2

NP = 10240
BN = 512
TW = 272
EP = 163840
SEG = 1024
NSEG = (EP // 16) // SEG
HALF = 5120
NHALF = 5000
TROWS = HALF // 16
CH = 128


def _mm_body(x_ref, wa_ref, wr_ref, t1_ref, t2_ref):
    x = x_ref[...]
    t1_ref[...] = jnp.dot(x, wa_ref[0], preferred_element_type=jnp.float32)
    t2_ref[...] = jnp.dot(x, wr_ref[0], preferred_element_type=jnp.float32)


def _feature_tables(inputs, conv_weights, attn_l, attn_r):
    xp = jnp.zeros((NP, IN), jnp.float32).at[:N].set(inputs)
    wfeat = conv_weights.transpose(0, 2, 1, 3).reshape(R, IN, H * DH)
    wl = jnp.einsum('rhid,rhd->rih', conv_weights, attn_l)
    wr = jnp.einsum('rhid,rhd->rih', conv_weights, attn_r)
    wa = jnp.concatenate([wfeat, wl, jnp.zeros((R, IN, 8), jnp.float32)], axis=-1)
    wb = jnp.concatenate([wr, jnp.zeros((R, IN, 8), jnp.float32)], axis=-1)
    nb = NP // BN
    t1, t2 = pl.pallas_call(
        _mm_body,
        grid=(R, nb),
        in_specs=[
            pl.BlockSpec((BN, IN), lambda r, i: (i, 0)),
            pl.BlockSpec((1, IN, TW), lambda r, i: (r, 0, 0)),
            pl.BlockSpec((1, IN, 16), lambda r, i: (r, 0, 0)),
        ],
        out_specs=[
            pl.BlockSpec((BN, TW), lambda r, i: (r * nb + i, 0)),
            pl.BlockSpec((BN, 16), lambda r, i: (r * nb + i, 0)),
        ],
        out_shape=[
            jax.ShapeDtypeStruct((R * NP, TW), jnp.float32),
            jax.ShapeDtypeStruct((R * NP, 16), jnp.float32),
        ],
    )(xp, wa, wb)
    return t1, t2


def _edge_body(t1, t2, colp, frow, fcol, zrows, out,
               acc, colb, frb, fcb, gb, eb, cb,
               featb, erb, sem1, sem2):
    cid = lax.axis_index("c")
    sid = lax.axis_index("s")
    lo = cid * NHALF
    hi = lo + NHALF

    pltpu.sync_copy(zrows, acc.at[pl.ds(sid * TROWS, TROWS)])
    plsc.subcore_barrier()

    ebase = sid * (EP // 16)
    trash16 = jnp.full((16,), NHALF, jnp.int32)
    zero16 = jnp.zeros((16,), jnp.int32)

    def seg_loop(s, _):
        base = ebase + s * SEG
        pltpu.sync_copy(colp.at[pl.ds(base, SEG)], colb)
        pltpu.sync_copy(frow.at[pl.ds(base, SEG)], frb)
        pltpu.sync_copy(fcol.at[pl.ds(base, SEG)], fcb)

        def fill(i, _):
            r = i >> 3
            q = pl.ds((i & 7) * 16, 16)
            cb.at[r][q] = trash16
            gb.at[r][q] = zero16
            eb.at[r][q] = zero16
            return 0
        lax.fori_loop(0, SEG // 16, fill, 0)

        def comp(i, off):
            c16 = colb[pl.ds(i * 16, 16)]
            m = (c16 >= lo) & (c16 < hi)
            mi = m.astype(jnp.int32)
            pos = off + plsc.cumsum(mi) - 1
            ph = pos >> 7
            pw = pos & (CH - 1)
            plsc.store_scatter(gb, [ph, pw], frb[pl.ds(i * 16, 16)], mask=m)
            plsc.store_scatter(eb, [ph, pw], fcb[pl.ds(i * 16, 16)], mask=m)
            plsc.store_scatter(cb, [ph, pw], c16 - lo, mask=m)
            return off + jnp.sum(mi)
        off = lax.fori_loop(0, SEG // 16, comp, jnp.int32(0))
        nch = (off + CH - 1) // CH

        def chunk(c, _):
            d1 = pltpu.async_copy(t1.at[gb.at[c]], featb, sem1)
            d2 = pltpu.async_copy(t2.at[eb.at[c]], erb, sem2)
            d1.wait()
            d2.wait()

            def ebody(e, _):
                frw = featb.at[e]
                el16 = frw[pl.ds(256, 16)]
                er16 = erb.at[e][pl.ds(0, 16)]
                s16 = el16 + er16
                s16 = jnp.where(s16 >= 0, s16, s16 * SLOPE)
                ee16 = jnp.exp(s16)
                frw[pl.ds(256, 16)] = ee16
                e16 = jnp.full((16,), e, jnp.int32)
                for h in range(H):
                    eh = plsc.load_gather(
                        featb, [e16, jnp.full((16,), 256 + h, jnp.int32)])
                    for k in (2 * h, 2 * h + 1):
                        frw[pl.ds(k * 16, 16)] = frw[pl.ds(k * 16, 16)] * eh
                return 0
            lax.fori_loop(0, CH, ebody, 0)
            pltpu.sync_copy(featb, acc.at[cb.at[c]], add=True)
            return 0
        lax.fori_loop(0, nch, chunk, 0)
        return 0
    lax.fori_loop(0, NSEG, seg_loop, 0)

    plsc.subcore_barrier()
    pltpu.sync_copy(acc.at[pl.ds(sid * TROWS, TROWS)],
                    out.at[cid, pl.ds(sid * TROWS, TROWS)])


def _edge_phase(t1, t2, colp, frow, fcol):
    zrows = jnp.zeros((TROWS, TW), jnp.float32)
    fn = pl.kernel(
        _edge_body,
        out_type=jax.ShapeDtypeStruct((2, HALF, TW), jnp.float32),
        mesh=plsc.VectorSubcoreMesh(core_axis_name="c", subcore_axis_name="s",
                                    num_cores=2, num_subcores=16),
        compiler_params=pltpu.CompilerParams(
            use_tc_tiling_on_sc=False, needs_layout_passes=False),
        scratch_types=[
            pltpu.VMEM_SHARED((HALF, TW), jnp.float32),
            pltpu.VMEM((SEG,), jnp.int32),
            pltpu.VMEM((SEG,), jnp.int32),
            pltpu.VMEM((SEG,), jnp.int32),
            pltpu.VMEM((SEG // CH, CH), jnp.int32),
            pltpu.VMEM((SEG // CH, CH), jnp.int32),
            pltpu.VMEM((SEG // CH, CH), jnp.int32),
            pltpu.VMEM((CH, TW), jnp.float32),
            pltpu.VMEM((CH, 16), jnp.float32),
            pltpu.SemaphoreType.DMA,
            pltpu.SemaphoreType.DMA,
        ],
    )
    return fn(t1, t2, colp, frow, fcol, zrows)


def _norm_body(acc_ref, b_ref, bias_ref, out_ref):
    a = acc_ref[0]
    msg = a[:, :256]
    den = a[:, 256:264]
    r = 1.0 / (den + 1e-16)
    rb = jnp.dot(r, b_ref[...], preferred_element_type=jnp.float32)
    out_ref[0] = msg * rb + bias_ref[...]


def _normalize(acc, h_bias):
    bmat = jnp.repeat(jnp.eye(8, dtype=jnp.float32), DH, axis=1)
    bias = h_bias.reshape(1, 256)
    nb = HALF // 512
    res = pl.pallas_call(
        _norm_body,
        grid=(2, nb),
        in_specs=[
            pl.BlockSpec((1, 512, TW), lambda c, i: (c, i, 0)),
            pl.BlockSpec((8, 256), lambda c, i: (0, 0)),
            pl.BlockSpec((1, 256), lambda c, i: (0, 0)),
        ],
        out_specs=pl.BlockSpec((1, 512, 256), lambda c, i: (c, i, 0)),
        out_shape=jax.ShapeDtypeStruct((2, HALF, 256), jnp.float32),
    )(acc, bmat, bias)
    return jnp.concatenate([res[0, :NHALF], res[1, :NHALF]], axis=0)


def kernel(inputs, row_indices, col_indices, edge_rel_sorted, conv_weights, attn_l, attn_r, h_bias):
    t1, t2 = _feature_tables(inputs, conv_weights, attn_l, attn_r)
    colp = jnp.full((EP,), -1, jnp.int32).at[:E].set(col_indices)
    frow = jnp.zeros((EP,), jnp.int32).at[:E].set(edge_rel_sorted * NP + row_indices)
    fcol = jnp.zeros((EP,), jnp.int32).at[:E].set(edge_rel_sorted * NP + col_indices)
    acc = _edge_phase(t1, t2, colp, frow, fcol)
    return _normalize(acc, h_bias)
